# Initial kernel scaffold; baseline (speedup 1.0000x reference)
import jax
import jax.numpy as jnp
from jax.experimental import pallas as pl


def kernel(input, table, Wih, Whh, bih, bhh, gamma, beta):
    raise NotImplementedError("write your pallas kernel here")


# speedup vs baseline: 9.3568x; 9.3568x over previous
import functools

import jax
import jax.numpy as jnp
from jax import lax
from jax.experimental import pallas as pl
from jax.experimental.pallas import tpu as pltpu
from jax.experimental.pallas import tpu_sc as plsc

T, B, H, V = 512, 8, 512, 32000
G4 = 4 * H
CHUNK = 128
NCHUNK = T // CHUNK
NC, NS = 2, 16
NW = NC * NS
ROWS_PER_W = (T * B) // NW


def _sc_gather_body(table_hbm, idx_hbm, out_hbm, idx_v, rows_v, sem):
    wid = lax.axis_index("s") * NC + lax.axis_index("c")
    base = wid * ROWS_PER_W
    pltpu.sync_copy(idx_hbm.at[pl.ds(base, ROWS_PER_W)], idx_v)
    pltpu.async_copy(table_hbm.at[idx_v], rows_v, sem).wait()
    pltpu.sync_copy(rows_v, out_hbm.at[pl.ds(base, ROWS_PER_W)])


@functools.cache
def _sc_gather():
    return functools.partial(
        pl.kernel,
        out_type=jax.ShapeDtypeStruct((T * B, H), jnp.float32),
        mesh=plsc.VectorSubcoreMesh(core_axis_name="c", subcore_axis_name="s"),
        scratch_types=[
            pltpu.VMEM((ROWS_PER_W,), jnp.int32),
            pltpu.VMEM((ROWS_PER_W, H), jnp.float32),
            pltpu.SemaphoreType.DMA,
        ],
    )(_sc_gather_body)


def _lstm_body(x_ref, tok_ref, wih_ref, whh_ref, bih_ref, bhh_ref,
               gam_ref, bet_ref, ctx_ref, hf_ref, cf_ref,
               outs1_ref, gates_ref):
    nlayer = wih_ref.shape[0]
    mask = (tok_ref[...] != 0).astype(jnp.int32)
    lengths = jnp.sum(mask, axis=0)
    len_col = lengths.reshape(B, 1)
    gam = gam_ref[...].reshape(1, H)
    bet = bet_ref[...].reshape(1, H)

    finals = []
    for l in range(nlayer):
        wih = wih_ref[l]
        whh = whh_ref[l]
        bsum = (bih_ref[l] + bhh_ref[l]).reshape(1, G4)
        h = jnp.zeros((B, H), jnp.float32)
        c = jnp.zeros((B, H), jnp.float32)
        last = l == nlayer - 1
        for ck in range(NCHUNK):
            base = ck * CHUNK * B
            if l == 0:
                xin = x_ref[pl.ds(base, CHUNK * B), :]
            else:
                xin = outs1_ref[pl.ds(base, CHUNK * B), :]
            gates_ref[...] = (
                jnp.dot(xin, wih, preferred_element_type=jnp.float32) + bsum)

            def step(ti, carry, _ck=ck, _whh=whh, _last=last):
                h, c = carry
                t = _ck * CHUNK + ti
                g = gates_ref[pl.ds(ti * B, B), :] + jnp.dot(
                    h, _whh, preferred_element_type=jnp.float32)
                i_g = jax.nn.sigmoid(g[:, 0:H])
                f_g = jax.nn.sigmoid(g[:, H:2 * H])
                g_g = jnp.tanh(g[:, 2 * H:3 * H])
                o_g = jax.nn.sigmoid(g[:, 3 * H:4 * H])
                cn = f_g * c + i_g * g_g
                hn = o_g * jnp.tanh(cn)
                valid = len_col > t
                hm = jnp.where(valid, hn, 0.0)
                if not _last:
                    outs1_ref[pl.ds(t * B, B), :] = hm
                else:
                    mu = jnp.mean(hn, axis=-1, keepdims=True)
                    var = jnp.mean((hn - mu) ** 2, axis=-1, keepdims=True)
                    ln = (hn - mu) * lax.rsqrt(var + 1e-5) * gam + bet
                    ctx_ref[pl.ds(t * B, B), :] = jnp.where(valid, ln, 0.0)
                return (jnp.where(valid, hn, h), jnp.where(valid, cn, c))

            h, c = lax.fori_loop(0, CHUNK, step, (h, c))
        finals.append((h, c))

    li = lengths[:, None]
    lj = lengths[None, :]
    ii = lax.broadcasted_iota(jnp.int32, (B, B), 0)
    jj = lax.broadcasted_iota(jnp.int32, (B, B), 1)
    before = jnp.logical_or(lj > li, jnp.logical_and(lj == li, jj < ii))
    rank = jnp.sum(before.astype(jnp.int32), axis=1)
    kk = lax.broadcasted_iota(jnp.int32, (B, B), 0)
    P = (rank[None, :] == kk).astype(jnp.float32)
    for l, (h, c) in enumerate(finals):
        hf_ref[l] = jnp.dot(P, h, preferred_element_type=jnp.float32)
        cf_ref[l] = jnp.dot(P, c, preferred_element_type=jnp.float32)


def _tc_lstm(x_flat, tokens, WihT, WhhT, bih, bhh, gamma, beta):
    nlayer = WihT.shape[0]
    return pl.pallas_call(
        _lstm_body,
        out_shape=[
            jax.ShapeDtypeStruct((T * B, H), jnp.float32),
            jax.ShapeDtypeStruct((nlayer, B, H), jnp.float32),
            jax.ShapeDtypeStruct((nlayer, B, H), jnp.float32),
        ],
        scratch_shapes=[
            pltpu.VMEM((T * B, H), jnp.float32),
            pltpu.VMEM((CHUNK * B, G4), jnp.float32),
        ],
        compiler_params=pltpu.CompilerParams(
            vmem_limit_bytes=120 * 1024 * 1024),
    )(x_flat, tokens, WihT, WhhT, bih, bhh, gamma, beta)


def kernel(input, table, Wih, Whh, bih, bhh, gamma, beta):
    idx_flat = input.reshape(-1)
    x_flat = _sc_gather()(table, idx_flat)
    WihT = jnp.swapaxes(Wih, 1, 2)
    WhhT = jnp.swapaxes(Whh, 1, 2)
    ctx, hf, cf = _tc_lstm(x_flat, input, WihT, WhhT, bih, bhh, gamma, beta)
    return ctx.reshape(T, B, H), hf, cf


# speedup vs baseline: 9.4771x; 1.0129x over previous
import functools

import jax
import jax.numpy as jnp
from jax import lax
from jax.experimental import pallas as pl
from jax.experimental.pallas import tpu as pltpu
from jax.experimental.pallas import tpu_sc as plsc

T, B, H, V = 512, 8, 512, 32000
G4 = 4 * H
CHUNK = 128
NCHUNK = T // CHUNK
NC, NS = 2, 16
NW = NC * NS
ROWS_PER_W = (T * B) // NW


def _sc_gather_body(table_hbm, idx_hbm, out_hbm, idx_v, rows_v, sem):
    wid = lax.axis_index("s") * NC + lax.axis_index("c")
    base = wid * ROWS_PER_W
    pltpu.sync_copy(idx_hbm.at[pl.ds(base, ROWS_PER_W)], idx_v)
    pltpu.async_copy(table_hbm.at[idx_v], rows_v, sem).wait()
    pltpu.sync_copy(rows_v, out_hbm.at[pl.ds(base, ROWS_PER_W)])


@functools.cache
def _sc_gather():
    return functools.partial(
        pl.kernel,
        out_type=jax.ShapeDtypeStruct((T * B, H), jnp.float32),
        mesh=plsc.VectorSubcoreMesh(core_axis_name="c", subcore_axis_name="s"),
        scratch_types=[
            pltpu.VMEM((ROWS_PER_W,), jnp.int32),
            pltpu.VMEM((ROWS_PER_W, H), jnp.float32),
            pltpu.SemaphoreType.DMA,
        ],
    )(_sc_gather_body)


def _lstm_body(x_ref, tok_ref, wih_ref, whh_ref, bih_ref, bhh_ref,
               gam_ref, bet_ref, ctx_ref, hf_ref, cf_ref,
               outs1_ref, gates_ref):
    nlayer = wih_ref.shape[0]
    mask = (tok_ref[...] != 0).astype(jnp.int32)
    lengths = jnp.sum(mask, axis=0)
    len_col = lengths.reshape(B, 1)
    gam = gam_ref[...].reshape(1, H)
    bet = bet_ref[...].reshape(1, H)

    finals = []
    for l in range(nlayer):
        wih = wih_ref[l]
        whh = whh_ref[l]
        bsum = (bih_ref[l] + bhh_ref[l]).reshape(1, G4)
        h = jnp.zeros((B, H), jnp.float32)
        c = jnp.zeros((B, H), jnp.float32)
        last = l == nlayer - 1
        for ck in range(NCHUNK):
            base = ck * CHUNK * B
            if l == 0:
                xin = x_ref[pl.ds(base, CHUNK * B), :]
            else:
                xin = outs1_ref[pl.ds(base, CHUNK * B), :]
            gates_ref[...] = (
                jnp.dot(xin, wih, preferred_element_type=jnp.float32) + bsum)

            def step(ti, carry, _ck=ck, _whh=whh, _last=last):
                h, c = carry
                t = _ck * CHUNK + ti
                g = gates_ref[pl.ds(ti * B, B), :] + jnp.dot(
                    h.astype(jnp.bfloat16), _whh,
                    preferred_element_type=jnp.float32)
                i_g = jax.nn.sigmoid(g[:, 0:H])
                f_g = jax.nn.sigmoid(g[:, H:2 * H])
                g_g = jnp.tanh(g[:, 2 * H:3 * H])
                o_g = jax.nn.sigmoid(g[:, 3 * H:4 * H])
                cn = f_g * c + i_g * g_g
                hn = o_g * jnp.tanh(cn)
                valid = len_col > t
                hm = jnp.where(valid, hn, 0.0)
                if not _last:
                    outs1_ref[pl.ds(t * B, B), :] = hm
                else:
                    mu = jnp.mean(hn, axis=-1, keepdims=True)
                    var = jnp.mean((hn - mu) ** 2, axis=-1, keepdims=True)
                    ln = (hn - mu) * lax.rsqrt(var + 1e-5) * gam + bet
                    ctx_ref[pl.ds(t * B, B), :] = jnp.where(valid, ln, 0.0)
                return (jnp.where(valid, hn, h), jnp.where(valid, cn, c))

            h, c = lax.fori_loop(0, CHUNK, step, (h, c))
        finals.append((h, c))

    li = lengths[:, None]
    lj = lengths[None, :]
    ii = lax.broadcasted_iota(jnp.int32, (B, B), 0)
    jj = lax.broadcasted_iota(jnp.int32, (B, B), 1)
    before = jnp.logical_or(lj > li, jnp.logical_and(lj == li, jj < ii))
    rank = jnp.sum(before.astype(jnp.int32), axis=1)
    kk = lax.broadcasted_iota(jnp.int32, (B, B), 0)
    P = (rank[None, :] == kk).astype(jnp.float32)
    for l, (h, c) in enumerate(finals):
        hf_ref[l] = jnp.dot(P, h, preferred_element_type=jnp.float32)
        cf_ref[l] = jnp.dot(P, c, preferred_element_type=jnp.float32)


def _tc_lstm(x_flat, tokens, WihT, WhhT, bih, bhh, gamma, beta):
    nlayer = WihT.shape[0]
    return pl.pallas_call(
        _lstm_body,
        out_shape=[
            jax.ShapeDtypeStruct((T * B, H), jnp.float32),
            jax.ShapeDtypeStruct((nlayer, B, H), jnp.float32),
            jax.ShapeDtypeStruct((nlayer, B, H), jnp.float32),
        ],
        scratch_shapes=[
            pltpu.VMEM((T * B, H), jnp.float32),
            pltpu.VMEM((CHUNK * B, G4), jnp.float32),
        ],
        compiler_params=pltpu.CompilerParams(
            vmem_limit_bytes=120 * 1024 * 1024),
    )(x_flat, tokens, WihT, WhhT, bih, bhh, gamma, beta)


def kernel(input, table, Wih, Whh, bih, bhh, gamma, beta):
    idx_flat = input.reshape(-1)
    x_flat = _sc_gather()(table, idx_flat)
    WihT = jnp.swapaxes(Wih, 1, 2)
    WhhT = jnp.swapaxes(Whh, 1, 2).astype(jnp.bfloat16)
    ctx, hf, cf = _tc_lstm(x_flat, input, WihT, WhhT, bih, bhh, gamma, beta)
    return ctx.reshape(T, B, H), hf, cf


# speedup vs baseline: 9.8303x; 1.0373x over previous
import functools

import jax
import jax.numpy as jnp
from jax import lax
from jax.experimental import pallas as pl
from jax.experimental.pallas import tpu as pltpu
from jax.experimental.pallas import tpu_sc as plsc

T, B, H, V = 512, 8, 512, 32000
G4 = 4 * H
CHUNK = 128
NCHUNK = T // CHUNK
NC, NS = 2, 16
NW = NC * NS
ROWS_PER_W = (T * B) // NW


def _sc_gather_body(table_hbm, idx_hbm, out_hbm, idx_v, rows_v, sem):
    wid = lax.axis_index("s") * NC + lax.axis_index("c")
    base = wid * ROWS_PER_W
    pltpu.sync_copy(idx_hbm.at[pl.ds(base, ROWS_PER_W)], idx_v)
    pltpu.async_copy(table_hbm.at[idx_v], rows_v, sem).wait()
    pltpu.sync_copy(rows_v, out_hbm.at[pl.ds(base, ROWS_PER_W)])


@functools.cache
def _sc_gather():
    return functools.partial(
        pl.kernel,
        out_type=jax.ShapeDtypeStruct((T * B, H), jnp.float32),
        mesh=plsc.VectorSubcoreMesh(core_axis_name="c", subcore_axis_name="s"),
        scratch_types=[
            pltpu.VMEM((ROWS_PER_W,), jnp.int32),
            pltpu.VMEM((ROWS_PER_W, H), jnp.float32),
            pltpu.SemaphoreType.DMA,
        ],
    )(_sc_gather_body)


def _gate_math(g, c):
    i_g = jax.nn.sigmoid(g[:, 0:H])
    f_g = jax.nn.sigmoid(g[:, H:2 * H])
    g_g = jnp.tanh(g[:, 2 * H:3 * H])
    o_g = jax.nn.sigmoid(g[:, 3 * H:4 * H])
    cn = f_g * c + i_g * g_g
    hn = o_g * jnp.tanh(cn)
    return hn, cn


def _lstm_body(x_ref, tok_ref, wih1_ref, whh1_ref, w2cat_ref, bih_ref,
               bhh_ref, gam_ref, bet_ref, ctx_ref, hf_ref, cf_ref,
               gates_ref):
    mask = (tok_ref[...] != 0).astype(jnp.int32)
    lengths = jnp.sum(mask, axis=0)
    len_col = lengths.reshape(B, 1)
    gam = gam_ref[...].reshape(1, H)
    bet = bet_ref[...].reshape(1, H)

    wih1 = wih1_ref[...]
    whh1 = whh1_ref[...]
    w2cat = w2cat_ref[...]
    bsum1 = (bih_ref[0] + bhh_ref[0]).reshape(1, G4)
    bsum2 = (bih_ref[1] + bhh_ref[1]).reshape(1, G4)

    def l2_step(t2, x2, h2, c2):
        cat = jnp.concatenate(
            [x2.astype(jnp.bfloat16), h2.astype(jnp.bfloat16)], axis=1)
        g = bsum2 + jnp.dot(cat, w2cat, preferred_element_type=jnp.float32)
        hn, cn = _gate_math(g, c2)
        valid = jnp.logical_and(len_col > t2,
                                jnp.broadcast_to(t2, (B, 1)) >= 0)
        mu = jnp.mean(hn, axis=-1, keepdims=True)
        var = jnp.mean((hn - mu) ** 2, axis=-1, keepdims=True)
        ln = (hn - mu) * lax.rsqrt(var + 1e-5) * gam + bet
        row = jnp.maximum(t2, 0) * B
        ctx_ref[pl.ds(row, B), :] = jnp.where(valid, ln, 0.0)
        return jnp.where(valid, hn, h2), jnp.where(valid, cn, c2)

    z = jnp.zeros((B, H), jnp.float32)
    carry = (z, z, z, z, z)
    for ck in range(NCHUNK):
        base = ck * CHUNK * B
        xin = x_ref[pl.ds(base, CHUNK * B), :]
        gates_ref[...] = (
            jnp.dot(xin, wih1, preferred_element_type=jnp.float32) + bsum1)

        def step(ti, carry, _ck=ck):
            h1, c1, h2, c2, x2 = carry
            t = _ck * CHUNK + ti
            h2n, c2n = l2_step(t - 1, x2, h2, c2)
            g1 = gates_ref[pl.ds(ti * B, B), :] + jnp.dot(
                h1.astype(jnp.bfloat16), whh1,
                preferred_element_type=jnp.float32)
            hn, cn = _gate_math(g1, c1)
            valid = len_col > t
            x2n = jnp.where(valid, hn, 0.0)
            return (jnp.where(valid, hn, h1), jnp.where(valid, cn, c1),
                    h2n, c2n, x2n)

        carry = lax.fori_loop(0, CHUNK, step, carry)
    h1, c1, h2, c2, x2 = carry
    h2, c2 = l2_step(T - 1, x2, h2, c2)

    li = lengths[:, None]
    lj = lengths[None, :]
    ii = lax.broadcasted_iota(jnp.int32, (B, B), 0)
    jj = lax.broadcasted_iota(jnp.int32, (B, B), 1)
    before = jnp.logical_or(lj > li, jnp.logical_and(lj == li, jj < ii))
    rank = jnp.sum(before.astype(jnp.int32), axis=1)
    kk = lax.broadcasted_iota(jnp.int32, (B, B), 0)
    P = (rank[None, :] == kk).astype(jnp.float32)
    for l, (h, c) in enumerate(((h1, c1), (h2, c2))):
        hf_ref[l] = jnp.dot(P, h, preferred_element_type=jnp.float32)
        cf_ref[l] = jnp.dot(P, c, preferred_element_type=jnp.float32)


def _tc_lstm(x_flat, tokens, Wih1T, Whh1T, W2cat, bih, bhh, gamma, beta):
    nlayer = bih.shape[0]
    return pl.pallas_call(
        _lstm_body,
        out_shape=[
            jax.ShapeDtypeStruct((T * B, H), jnp.float32),
            jax.ShapeDtypeStruct((nlayer, B, H), jnp.float32),
            jax.ShapeDtypeStruct((nlayer, B, H), jnp.float32),
        ],
        scratch_shapes=[
            pltpu.VMEM((CHUNK * B, G4), jnp.float32),
        ],
        compiler_params=pltpu.CompilerParams(
            vmem_limit_bytes=120 * 1024 * 1024),
    )(x_flat, tokens, Wih1T, Whh1T, W2cat, bih, bhh, gamma, beta)


def kernel(input, table, Wih, Whh, bih, bhh, gamma, beta):
    idx_flat = input.reshape(-1)
    x_flat = _sc_gather()(table, idx_flat)
    WihT = jnp.swapaxes(Wih, 1, 2)
    WhhT = jnp.swapaxes(Whh, 1, 2)
    W2cat = jnp.concatenate([WihT[1], WhhT[1]], axis=0).astype(jnp.bfloat16)
    ctx, hf, cf = _tc_lstm(x_flat, input, WihT[0],
                           WhhT[0].astype(jnp.bfloat16), W2cat,
                           bih, bhh, gamma, beta)
    return ctx.reshape(T, B, H), hf, cf


# speedup vs baseline: 10.0329x; 1.0206x over previous
import functools

import jax
import jax.numpy as jnp
from jax import lax
from jax.experimental import pallas as pl
from jax.experimental.pallas import tpu as pltpu
from jax.experimental.pallas import tpu_sc as plsc

T, B, H, V = 512, 8, 512, 32000
G4 = 4 * H
CHUNK = 128
NCHUNK = T // CHUNK
NC, NS = 2, 16
NW = NC * NS
ROWS_PER_W = (T * B) // NW


def _sc_gather_body(table_hbm, idx_hbm, out_hbm, idx_v, rows_v, sem):
    wid = lax.axis_index("s") * NC + lax.axis_index("c")
    base = wid * ROWS_PER_W
    pltpu.sync_copy(idx_hbm.at[pl.ds(base, ROWS_PER_W)], idx_v)
    pltpu.async_copy(table_hbm.at[idx_v], rows_v, sem).wait()
    pltpu.sync_copy(rows_v, out_hbm.at[pl.ds(base, ROWS_PER_W)])


@functools.cache
def _sc_gather():
    return functools.partial(
        pl.kernel,
        out_type=jax.ShapeDtypeStruct((T * B, H), jnp.float32),
        mesh=plsc.VectorSubcoreMesh(core_axis_name="c", subcore_axis_name="s"),
        scratch_types=[
            pltpu.VMEM((ROWS_PER_W,), jnp.int32),
            pltpu.VMEM((ROWS_PER_W, H), jnp.float32),
            pltpu.SemaphoreType.DMA,
        ],
    )(_sc_gather_body)


def _gate_math(g, c):
    i_g = jax.nn.sigmoid(g[:, 0:H])
    f_g = jax.nn.sigmoid(g[:, H:2 * H])
    g_g = jnp.tanh(g[:, 2 * H:3 * H])
    o_g = jax.nn.sigmoid(g[:, 3 * H:4 * H])
    cn = f_g * c + i_g * g_g
    hn = o_g * jnp.tanh(cn)
    return hn, cn


def _lstm_body(x_ref, tok_ref, wih1_ref, whh1_ref, w2cat_ref, bih_ref,
               bhh_ref, gam_ref, bet_ref, ctx_ref, hf_ref, cf_ref,
               gates_ref):
    mask = (tok_ref[...] != 0).astype(jnp.int32)
    lengths = jnp.sum(mask, axis=0)
    len_col = lengths.reshape(B, 1)
    gam = gam_ref[...].reshape(1, H)
    bet = bet_ref[...].reshape(1, H)

    wih1 = wih1_ref[...]
    whh1 = whh1_ref[...]
    w2cat = w2cat_ref[...]
    bsum1 = (bih_ref[0] + bhh_ref[0]).reshape(1, G4)
    bsum2 = (bih_ref[1] + bhh_ref[1]).reshape(1, G4)

    def l2_step(t2, x2, h2, c2):
        cat = jnp.concatenate(
            [x2.astype(jnp.bfloat16), h2.astype(jnp.bfloat16)], axis=1)
        g = bsum2 + jnp.dot(cat, w2cat, preferred_element_type=jnp.float32)
        hn, cn = _gate_math(g, c2)
        valid = jnp.logical_and(len_col > t2,
                                jnp.broadcast_to(t2, (B, 1)) >= 0)
        mu = jnp.mean(hn, axis=-1, keepdims=True)
        var = jnp.mean((hn - mu) ** 2, axis=-1, keepdims=True)
        ln = (hn - mu) * lax.rsqrt(var + 1e-5) * gam + bet
        row = jnp.maximum(t2, 0) * B
        ctx_ref[pl.ds(row, B), :] = jnp.where(valid, ln, 0.0)
        return jnp.where(valid, hn, h2), jnp.where(valid, cn, c2)

    z = jnp.zeros((B, H), jnp.float32)
    carry = (z, z, z, z, z)
    for ck in range(NCHUNK):
        base = ck * CHUNK * B
        xin = x_ref[pl.ds(base, CHUNK * B), :]
        gates_ref[...] = (
            jnp.dot(xin, wih1, preferred_element_type=jnp.float32) + bsum1)

        def step(ti, carry, _ck=ck):
            h1, c1, h2, c2, x2 = carry
            t = _ck * CHUNK + ti
            h2n, c2n = l2_step(t - 1, x2, h2, c2)
            g1 = gates_ref[pl.ds(ti * B, B), :] + jnp.dot(
                h1.astype(jnp.bfloat16), whh1,
                preferred_element_type=jnp.float32)
            hn, cn = _gate_math(g1, c1)
            valid = len_col > t
            x2n = jnp.where(valid, hn, 0.0)
            return (jnp.where(valid, hn, h1), jnp.where(valid, cn, c1),
                    h2n, c2n, x2n)

        carry = lax.fori_loop(0, CHUNK, step, carry, unroll=2)
    h1, c1, h2, c2, x2 = carry
    h2, c2 = l2_step(T - 1, x2, h2, c2)

    li = lengths[:, None]
    lj = lengths[None, :]
    ii = lax.broadcasted_iota(jnp.int32, (B, B), 0)
    jj = lax.broadcasted_iota(jnp.int32, (B, B), 1)
    before = jnp.logical_or(lj > li, jnp.logical_and(lj == li, jj < ii))
    rank = jnp.sum(before.astype(jnp.int32), axis=1)
    kk = lax.broadcasted_iota(jnp.int32, (B, B), 0)
    P = (rank[None, :] == kk).astype(jnp.float32)
    for l, (h, c) in enumerate(((h1, c1), (h2, c2))):
        hf_ref[l] = jnp.dot(P, h, preferred_element_type=jnp.float32)
        cf_ref[l] = jnp.dot(P, c, preferred_element_type=jnp.float32)


def _tc_lstm(x_flat, tokens, Wih1T, Whh1T, W2cat, bih, bhh, gamma, beta):
    nlayer = bih.shape[0]
    return pl.pallas_call(
        _lstm_body,
        out_shape=[
            jax.ShapeDtypeStruct((T * B, H), jnp.float32),
            jax.ShapeDtypeStruct((nlayer, B, H), jnp.float32),
            jax.ShapeDtypeStruct((nlayer, B, H), jnp.float32),
        ],
        scratch_shapes=[
            pltpu.VMEM((CHUNK * B, G4), jnp.float32),
        ],
        compiler_params=pltpu.CompilerParams(
            vmem_limit_bytes=120 * 1024 * 1024),
    )(x_flat, tokens, Wih1T, Whh1T, W2cat, bih, bhh, gamma, beta)


def kernel(input, table, Wih, Whh, bih, bhh, gamma, beta):
    idx_flat = input.reshape(-1)
    x_flat = _sc_gather()(table, idx_flat)
    WihT = jnp.swapaxes(Wih, 1, 2)
    WhhT = jnp.swapaxes(Whh, 1, 2)
    W2cat = jnp.concatenate([WihT[1], WhhT[1]], axis=0).astype(jnp.bfloat16)
    ctx, hf, cf = _tc_lstm(x_flat, input, WihT[0],
                           WhhT[0].astype(jnp.bfloat16), W2cat,
                           bih, bhh, gamma, beta)
    return ctx.reshape(T, B, H), hf, cf


# speedup vs baseline: 10.0466x; 1.0014x over previous
import functools

import jax
import jax.numpy as jnp
from jax import lax
from jax.experimental import pallas as pl
from jax.experimental.pallas import tpu as pltpu
from jax.experimental.pallas import tpu_sc as plsc

T, B, H, V = 512, 8, 512, 32000
G4 = 4 * H
CHUNK = 128
NCHUNK = T // CHUNK
NC, NS = 2, 16
NW = NC * NS
ROWS_PER_W = (T * B) // NW


def _sc_gather_body(table_hbm, idx_hbm, out_hbm, idx_v, rows_v, sem):
    wid = lax.axis_index("s") * NC + lax.axis_index("c")
    base = wid * ROWS_PER_W
    pltpu.sync_copy(idx_hbm.at[pl.ds(base, ROWS_PER_W)], idx_v)
    pltpu.async_copy(table_hbm.at[idx_v], rows_v, sem).wait()
    pltpu.sync_copy(rows_v, out_hbm.at[pl.ds(base, ROWS_PER_W)])


@functools.cache
def _sc_gather():
    return functools.partial(
        pl.kernel,
        out_type=jax.ShapeDtypeStruct((T * B, H), jnp.float32),
        mesh=plsc.VectorSubcoreMesh(core_axis_name="c", subcore_axis_name="s"),
        scratch_types=[
            pltpu.VMEM((ROWS_PER_W,), jnp.int32),
            pltpu.VMEM((ROWS_PER_W, H), jnp.float32),
            pltpu.SemaphoreType.DMA,
        ],
    )(_sc_gather_body)


def _gate_math(g, c):
    i_g = jax.nn.sigmoid(g[:, 0:H])
    f_g = jax.nn.sigmoid(g[:, H:2 * H])
    g_g = jnp.tanh(g[:, 2 * H:3 * H])
    o_g = jax.nn.sigmoid(g[:, 3 * H:4 * H])
    cn = f_g * c + i_g * g_g
    hn = o_g * jnp.tanh(cn)
    return hn, cn


def _lstm_body(x_ref, tok_ref, wih1_ref, whh1_ref, w2cat_ref, bih_ref,
               bhh_ref, gam_ref, bet_ref, ctx_ref, hf_ref, cf_ref,
               gates_ref):
    mask = (tok_ref[...] != 0).astype(jnp.int32)
    lengths = jnp.sum(mask, axis=0)
    len_col = lengths.reshape(B, 1)
    gam = gam_ref[...].reshape(1, H)
    bet = bet_ref[...].reshape(1, H)

    wih1 = wih1_ref[...]
    whh1 = whh1_ref[...]
    w2cat = w2cat_ref[...]
    bsum1 = (bih_ref[0] + bhh_ref[0]).reshape(1, G4)
    bsum2 = (bih_ref[1] + bhh_ref[1]).reshape(1, G4)

    def ln_store(tln, y2):
        s1 = jnp.sum(y2, axis=-1, keepdims=True)
        s2 = jnp.sum(y2 * y2, axis=-1, keepdims=True)
        mu = s1 * (1.0 / H)
        var = s2 * (1.0 / H) - mu * mu
        ln = (y2 - mu) * lax.rsqrt(var + 1e-5) * gam + bet
        valid = jnp.logical_and(len_col > tln,
                                jnp.broadcast_to(tln, (B, 1)) >= 0)
        row = pl.multiple_of(jnp.maximum(tln, 0) * B, B)
        ctx_ref[pl.ds(row, B), :] = jnp.where(valid, ln, 0.0)

    def l2_step(t2, x2, h2, c2):
        cat = jnp.concatenate(
            [x2.astype(jnp.bfloat16), h2.astype(jnp.bfloat16)], axis=1)
        g = bsum2 + jnp.dot(cat, w2cat, preferred_element_type=jnp.float32)
        hn, cn = _gate_math(g, c2)
        valid = jnp.logical_and(len_col > t2,
                                jnp.broadcast_to(t2, (B, 1)) >= 0)
        y2 = jnp.where(valid, hn, 0.0)
        return jnp.where(valid, hn, h2), jnp.where(valid, cn, c2), y2

    z = jnp.zeros((B, H), jnp.float32)
    carry = (z, z, z, z, z, z)
    for ck in range(NCHUNK):
        base = ck * CHUNK * B
        xin = x_ref[pl.ds(base, CHUNK * B), :]
        gates_ref[...] = (
            jnp.dot(xin, wih1, preferred_element_type=jnp.float32) + bsum1)

        def step(ti, carry, _ck=ck):
            h1, c1, h2, c2, x2, y2 = carry
            t = _ck * CHUNK + ti
            ln_store(t - 2, y2)
            h2n, c2n, y2n = l2_step(t - 1, x2, h2, c2)
            g1 = gates_ref[pl.ds(pl.multiple_of(ti * B, B), B), :] + jnp.dot(
                h1.astype(jnp.bfloat16), whh1,
                preferred_element_type=jnp.float32)
            hn, cn = _gate_math(g1, c1)
            valid = len_col > t
            x2n = jnp.where(valid, hn, 0.0)
            return (jnp.where(valid, hn, h1), jnp.where(valid, cn, c1),
                    h2n, c2n, x2n, y2n)

        carry = lax.fori_loop(0, CHUNK, step, carry, unroll=2)
    h1, c1, h2, c2, x2, y2 = carry
    ln_store(T - 2, y2)
    h2, c2, y2 = l2_step(T - 1, x2, h2, c2)
    ln_store(T - 1, y2)

    li = lengths[:, None]
    lj = lengths[None, :]
    ii = lax.broadcasted_iota(jnp.int32, (B, B), 0)
    jj = lax.broadcasted_iota(jnp.int32, (B, B), 1)
    before = jnp.logical_or(lj > li, jnp.logical_and(lj == li, jj < ii))
    rank = jnp.sum(before.astype(jnp.int32), axis=1)
    kk = lax.broadcasted_iota(jnp.int32, (B, B), 0)
    P = (rank[None, :] == kk).astype(jnp.float32)
    for l, (h, c) in enumerate(((h1, c1), (h2, c2))):
        hf_ref[l] = jnp.dot(P, h, preferred_element_type=jnp.float32)
        cf_ref[l] = jnp.dot(P, c, preferred_element_type=jnp.float32)


def _tc_lstm(x_flat, tokens, Wih1T, Whh1T, W2cat, bih, bhh, gamma, beta):
    nlayer = bih.shape[0]
    return pl.pallas_call(
        _lstm_body,
        out_shape=[
            jax.ShapeDtypeStruct((T * B, H), jnp.float32),
            jax.ShapeDtypeStruct((nlayer, B, H), jnp.float32),
            jax.ShapeDtypeStruct((nlayer, B, H), jnp.float32),
        ],
        scratch_shapes=[
            pltpu.VMEM((CHUNK * B, G4), jnp.float32),
        ],
        compiler_params=pltpu.CompilerParams(
            vmem_limit_bytes=120 * 1024 * 1024),
    )(x_flat, tokens, Wih1T, Whh1T, W2cat, bih, bhh, gamma, beta)


def kernel(input, table, Wih, Whh, bih, bhh, gamma, beta):
    idx_flat = input.reshape(-1)
    x_flat = _sc_gather()(table, idx_flat)
    WihT = jnp.swapaxes(Wih, 1, 2)
    WhhT = jnp.swapaxes(Whh, 1, 2)
    W2cat = jnp.concatenate([WihT[1], WhhT[1]], axis=0).astype(jnp.bfloat16)
    ctx, hf, cf = _tc_lstm(x_flat, input, WihT[0],
                           WhhT[0].astype(jnp.bfloat16), W2cat,
                           bih, bhh, gamma, beta)
    return ctx.reshape(T, B, H), hf, cf


# speedup vs baseline: 12.0000x; 1.1944x over previous
import functools

import jax
import jax.numpy as jnp
from jax import lax
from jax.experimental import pallas as pl
from jax.experimental.pallas import tpu as pltpu
from jax.experimental.pallas import tpu_sc as plsc

T, B, H, V = 512, 8, 512, 32000
G4 = 4 * H
CHUNK = 128
NCHUNK = T // CHUNK
NC, NS = 2, 16
NW = NC * NS
ROWS_PER_W = (T * B) // NW


def _sc_gather_body(table_hbm, idx_hbm, out_hbm, idx_v, rows_v, sem):
    wid = lax.axis_index("s") * NC + lax.axis_index("c")
    base = wid * ROWS_PER_W
    pltpu.sync_copy(idx_hbm.at[pl.ds(base, ROWS_PER_W)], idx_v)
    pltpu.async_copy(table_hbm.at[idx_v], rows_v, sem).wait()
    pltpu.sync_copy(rows_v, out_hbm.at[pl.ds(base, ROWS_PER_W)])


@functools.cache
def _sc_gather():
    return functools.partial(
        pl.kernel,
        out_type=jax.ShapeDtypeStruct((T * B, H), jnp.float32),
        mesh=plsc.VectorSubcoreMesh(core_axis_name="c", subcore_axis_name="s"),
        scratch_types=[
            pltpu.VMEM((ROWS_PER_W,), jnp.int32),
            pltpu.VMEM((ROWS_PER_W, H), jnp.float32),
            pltpu.SemaphoreType.DMA,
        ],
    )(_sc_gather_body)


def _gate_math(g, c):
    i_g = jax.nn.sigmoid(g[:, 0:H])
    f_g = jax.nn.sigmoid(g[:, H:2 * H])
    g_g = jnp.tanh(g[:, 2 * H:3 * H])
    o_g = jax.nn.sigmoid(g[:, 3 * H:4 * H])
    cn = f_g * c + i_g * g_g
    hn = o_g * jnp.tanh(cn)
    return hn, cn


def _lstm_body(x_ref, tok_ref, wih1_ref, whh1_ref, wih2_ref, whh2_ref,
               bih_ref, bhh_ref, gam_ref, bet_ref, ctx_ref, hf_ref, cf_ref,
               gates1_ref, gates2_ref, outs1_ref):
    mask = (tok_ref[...] != 0).astype(jnp.int32)
    lengths = jnp.sum(mask, axis=0)
    len_col = lengths.reshape(B, 1)
    gam = gam_ref[...].reshape(1, H)
    bet = bet_ref[...].reshape(1, H)

    wih1 = wih1_ref[...]
    whh1 = whh1_ref[...]
    wih2 = wih2_ref[...]
    whh2 = whh2_ref[...]
    bsum1 = (bih_ref[0] + bhh_ref[0]).reshape(1, G4)
    bsum2 = (bih_ref[1] + bhh_ref[1]).reshape(1, G4)

    def l1_step(t, ti, h1, c1):
        g = gates1_ref[pl.ds(pl.multiple_of(ti * B, B), B), :] + jnp.dot(
            h1.astype(jnp.bfloat16), whh1, preferred_element_type=jnp.float32)
        hn, cn = _gate_math(g, c1)
        valid = len_col > t
        outs1_ref[pl.ds(pl.multiple_of(ti * B, B), B), :] = (
            jnp.where(valid, hn, 0.0))
        return jnp.where(valid, hn, h1), jnp.where(valid, cn, c1)

    def l2_step(t2, ti, row, h2, c2):
        g = gates2_ref[pl.ds(pl.multiple_of(ti * B, B), B), :]
        g = g + jnp.dot(h2.astype(jnp.bfloat16), whh2,
                        preferred_element_type=jnp.float32)
        hn, cn = _gate_math(g, c2)
        valid = len_col > t2
        y2 = jnp.where(valid, hn, 0.0)
        s1 = jnp.sum(y2, axis=-1, keepdims=True)
        s2 = jnp.sum(y2 * y2, axis=-1, keepdims=True)
        mu = s1 * (1.0 / H)
        var = s2 * (1.0 / H) - mu * mu
        ln = (y2 - mu) * lax.rsqrt(var + 1e-5) * gam + bet
        ctx_ref[pl.ds(row, B), :] = jnp.where(valid, ln, 0.0)
        return jnp.where(valid, hn, h2), jnp.where(valid, cn, c2)

    z = jnp.zeros((B, H), jnp.float32)
    carry = (z, z, z, z)
    for ph in range(NCHUNK + 1):
        if ph < NCHUNK:
            xin = x_ref[pl.ds(ph * CHUNK * B, CHUNK * B), :]
            gates1_ref[...] = (
                jnp.dot(xin, wih1, preferred_element_type=jnp.float32)
                + bsum1)
        if ph > 0:
            gates2_ref[...] = (
                jnp.dot(outs1_ref[...], wih2,
                        preferred_element_type=jnp.float32) + bsum2)

        def step(ti, carry, _ph=ph):
            h1, c1, h2, c2 = carry
            if _ph < NCHUNK:
                h1, c1 = l1_step(_ph * CHUNK + ti, ti, h1, c1)
            if _ph > 0:
                t2 = (_ph - 1) * CHUNK + ti
                row = pl.multiple_of((_ph - 1) * CHUNK * B + ti * B, B)
                h2, c2 = l2_step(t2, ti, row, h2, c2)
            return h1, c1, h2, c2

        carry = lax.fori_loop(0, CHUNK, step, carry, unroll=2)
    h1, c1, h2, c2 = carry

    li = lengths[:, None]
    lj = lengths[None, :]
    ii = lax.broadcasted_iota(jnp.int32, (B, B), 0)
    jj = lax.broadcasted_iota(jnp.int32, (B, B), 1)
    before = jnp.logical_or(lj > li, jnp.logical_and(lj == li, jj < ii))
    rank = jnp.sum(before.astype(jnp.int32), axis=1)
    kk = lax.broadcasted_iota(jnp.int32, (B, B), 0)
    P = (rank[None, :] == kk).astype(jnp.float32)
    for l, (h, c) in enumerate(((h1, c1), (h2, c2))):
        hf_ref[l] = jnp.dot(P, h, preferred_element_type=jnp.float32)
        cf_ref[l] = jnp.dot(P, c, preferred_element_type=jnp.float32)


def _tc_lstm(x_flat, tokens, Wih1T, Whh1T, Wih2T, Whh2T, bih, bhh,
             gamma, beta):
    nlayer = bih.shape[0]
    return pl.pallas_call(
        _lstm_body,
        out_shape=[
            jax.ShapeDtypeStruct((T * B, H), jnp.float32),
            jax.ShapeDtypeStruct((nlayer, B, H), jnp.float32),
            jax.ShapeDtypeStruct((nlayer, B, H), jnp.float32),
        ],
        scratch_shapes=[
            pltpu.VMEM((CHUNK * B, G4), jnp.float32),
            pltpu.VMEM((CHUNK * B, G4), jnp.float32),
            pltpu.VMEM((CHUNK * B, H), jnp.float32),
        ],
        compiler_params=pltpu.CompilerParams(
            vmem_limit_bytes=120 * 1024 * 1024),
    )(x_flat, tokens, Wih1T, Whh1T, Wih2T, Whh2T, bih, bhh, gamma, beta)


def kernel(input, table, Wih, Whh, bih, bhh, gamma, beta):
    idx_flat = input.reshape(-1)
    x_flat = _sc_gather()(table, idx_flat)
    WihT = jnp.swapaxes(Wih, 1, 2)
    WhhT = jnp.swapaxes(Whh, 1, 2)
    ctx, hf, cf = _tc_lstm(x_flat, input, WihT[0],
                           WhhT[0].astype(jnp.bfloat16), WihT[1],
                           WhhT[1].astype(jnp.bfloat16),
                           bih, bhh, gamma, beta)
    return ctx.reshape(T, B, H), hf, cf


# speedup vs baseline: 12.9891x; 1.0824x over previous
import functools

import jax
import jax.numpy as jnp
from jax import lax
from jax.experimental import pallas as pl
from jax.experimental.pallas import tpu as pltpu
from jax.experimental.pallas import tpu_sc as plsc

T, B, H, V = 512, 8, 512, 32000
G4 = 4 * H
CHUNK = 64
NCHUNK = T // CHUNK
NC, NS = 2, 16
NW = NC * NS
ROWS_PER_W = (T * B) // NW


def _sc_gather_body(table_hbm, idx_hbm, out_hbm, idx_v, rows_v, sem):
    wid = lax.axis_index("s") * NC + lax.axis_index("c")
    base = wid * ROWS_PER_W
    pltpu.sync_copy(idx_hbm.at[pl.ds(base, ROWS_PER_W)], idx_v)
    pltpu.async_copy(table_hbm.at[idx_v], rows_v, sem).wait()
    pltpu.sync_copy(rows_v, out_hbm.at[pl.ds(base, ROWS_PER_W)])


@functools.cache
def _sc_gather():
    return functools.partial(
        pl.kernel,
        out_type=jax.ShapeDtypeStruct((T * B, H), jnp.float32),
        mesh=plsc.VectorSubcoreMesh(core_axis_name="c", subcore_axis_name="s"),
        scratch_types=[
            pltpu.VMEM((ROWS_PER_W,), jnp.int32),
            pltpu.VMEM((ROWS_PER_W, H), jnp.float32),
            pltpu.SemaphoreType.DMA,
        ],
    )(_sc_gather_body)


def _gate_math(g, c):
    i_g = jax.nn.sigmoid(g[:, 0:H])
    f_g = jax.nn.sigmoid(g[:, H:2 * H])
    g_g = jnp.tanh(g[:, 2 * H:3 * H])
    o_g = jax.nn.sigmoid(g[:, 3 * H:4 * H])
    cn = f_g * c + i_g * g_g
    hn = o_g * jnp.tanh(cn)
    return hn, cn


def _lstm_body(x_ref, tok_ref, wih1_ref, whh1_ref, wih2_ref, whh2_ref,
               bih_ref, bhh_ref, gam_ref, bet_ref, ctx_ref, hf_ref, cf_ref,
               gates1_ref, gates2_ref, outs1_ref):
    mask = (tok_ref[...] != 0).astype(jnp.int32)
    lengths = jnp.sum(mask, axis=0)
    len_col = lengths.reshape(B, 1)
    gam = gam_ref[...].reshape(1, H)
    bet = bet_ref[...].reshape(1, H)

    wih1 = wih1_ref[...]
    whh1 = whh1_ref[...]
    wih2 = wih2_ref[...]
    whh2 = whh2_ref[...]
    bsum1 = (bih_ref[0] + bhh_ref[0]).reshape(1, G4)
    bsum2 = (bih_ref[1] + bhh_ref[1]).reshape(1, G4)

    def l1_step(t, ti, h1, c1):
        g = gates1_ref[pl.ds(pl.multiple_of(ti * B, B), B), :] + jnp.dot(
            h1.astype(jnp.bfloat16), whh1, preferred_element_type=jnp.float32)
        hn, cn = _gate_math(g, c1)
        valid = len_col > t
        outs1_ref[pl.ds(pl.multiple_of(ti * B, B), B), :] = (
            jnp.where(valid, hn, 0.0))
        return jnp.where(valid, hn, h1), jnp.where(valid, cn, c1)

    def l2_step(t2, ti, row, h2, c2):
        g = gates2_ref[pl.ds(pl.multiple_of(ti * B, B), B), :]
        g = g + jnp.dot(h2.astype(jnp.bfloat16), whh2,
                        preferred_element_type=jnp.float32)
        hn, cn = _gate_math(g, c2)
        valid = len_col > t2
        y2 = jnp.where(valid, hn, 0.0)
        s1 = jnp.sum(y2, axis=-1, keepdims=True)
        s2 = jnp.sum(y2 * y2, axis=-1, keepdims=True)
        mu = s1 * (1.0 / H)
        var = s2 * (1.0 / H) - mu * mu
        ln = (y2 - mu) * lax.rsqrt(var + 1e-5) * gam + bet
        ctx_ref[pl.ds(row, B), :] = jnp.where(valid, ln, 0.0)
        return jnp.where(valid, hn, h2), jnp.where(valid, cn, c2)

    z = jnp.zeros((B, H), jnp.float32)
    carry = (z, z, z, z)
    for ph in range(NCHUNK + 1):
        if ph < NCHUNK:
            xin = x_ref[pl.ds(ph * CHUNK * B, CHUNK * B), :]
            gates1_ref[...] = (
                jnp.dot(xin, wih1, preferred_element_type=jnp.float32)
                + bsum1)
        if ph > 0:
            gates2_ref[...] = (
                jnp.dot(outs1_ref[...], wih2,
                        preferred_element_type=jnp.float32) + bsum2)

        def step(ti, carry, _ph=ph):
            h1, c1, h2, c2 = carry
            if _ph < NCHUNK:
                h1, c1 = l1_step(_ph * CHUNK + ti, ti, h1, c1)
            if _ph > 0:
                t2 = (_ph - 1) * CHUNK + ti
                row = pl.multiple_of((_ph - 1) * CHUNK * B + ti * B, B)
                h2, c2 = l2_step(t2, ti, row, h2, c2)
            return h1, c1, h2, c2

        carry = lax.fori_loop(0, CHUNK, step, carry, unroll=2)
    h1, c1, h2, c2 = carry

    li = lengths[:, None]
    lj = lengths[None, :]
    ii = lax.broadcasted_iota(jnp.int32, (B, B), 0)
    jj = lax.broadcasted_iota(jnp.int32, (B, B), 1)
    before = jnp.logical_or(lj > li, jnp.logical_and(lj == li, jj < ii))
    rank = jnp.sum(before.astype(jnp.int32), axis=1)
    kk = lax.broadcasted_iota(jnp.int32, (B, B), 0)
    P = (rank[None, :] == kk).astype(jnp.float32)
    for l, (h, c) in enumerate(((h1, c1), (h2, c2))):
        hf_ref[l] = jnp.dot(P, h, preferred_element_type=jnp.float32)
        cf_ref[l] = jnp.dot(P, c, preferred_element_type=jnp.float32)


def _tc_lstm(x_flat, tokens, Wih1T, Whh1T, Wih2T, Whh2T, bih, bhh,
             gamma, beta):
    nlayer = bih.shape[0]
    return pl.pallas_call(
        _lstm_body,
        out_shape=[
            jax.ShapeDtypeStruct((T * B, H), jnp.float32),
            jax.ShapeDtypeStruct((nlayer, B, H), jnp.float32),
            jax.ShapeDtypeStruct((nlayer, B, H), jnp.float32),
        ],
        scratch_shapes=[
            pltpu.VMEM((CHUNK * B, G4), jnp.float32),
            pltpu.VMEM((CHUNK * B, G4), jnp.float32),
            pltpu.VMEM((CHUNK * B, H), jnp.float32),
        ],
        compiler_params=pltpu.CompilerParams(
            vmem_limit_bytes=120 * 1024 * 1024),
    )(x_flat, tokens, Wih1T, Whh1T, Wih2T, Whh2T, bih, bhh, gamma, beta)


def kernel(input, table, Wih, Whh, bih, bhh, gamma, beta):
    idx_flat = input.reshape(-1)
    x_flat = _sc_gather()(table, idx_flat)
    WihT = jnp.swapaxes(Wih, 1, 2)
    WhhT = jnp.swapaxes(Whh, 1, 2)
    ctx, hf, cf = _tc_lstm(x_flat, input, WihT[0],
                           WhhT[0].astype(jnp.bfloat16), WihT[1],
                           WhhT[1].astype(jnp.bfloat16),
                           bih, bhh, gamma, beta)
    return ctx.reshape(T, B, H), hf, cf


# speedup vs baseline: 13.9409x; 1.0733x over previous
import functools

import jax
import jax.numpy as jnp
from jax import lax
from jax.experimental import pallas as pl
from jax.experimental.pallas import tpu as pltpu
from jax.experimental.pallas import tpu_sc as plsc

T, B, H, V = 512, 8, 512, 32000
G4 = 4 * H
CHUNK = 64
NCHUNK = T // CHUNK
NC, NS = 2, 16
NW = NC * NS
ROWS_PER_W = (T * B) // NW


def _sc_gather_body(table_hbm, idx_hbm, out_hbm, idx_v, rows_v, sem):
    wid = lax.axis_index("s") * NC + lax.axis_index("c")
    base = wid * ROWS_PER_W
    pltpu.sync_copy(idx_hbm.at[pl.ds(base, ROWS_PER_W)], idx_v)
    pltpu.async_copy(table_hbm.at[idx_v], rows_v, sem).wait()
    pltpu.sync_copy(rows_v, out_hbm.at[pl.ds(base, ROWS_PER_W)])


@functools.cache
def _sc_gather():
    return functools.partial(
        pl.kernel,
        out_type=jax.ShapeDtypeStruct((T * B, H), jnp.float32),
        mesh=plsc.VectorSubcoreMesh(core_axis_name="c", subcore_axis_name="s"),
        scratch_types=[
            pltpu.VMEM((ROWS_PER_W,), jnp.int32),
            pltpu.VMEM((ROWS_PER_W, H), jnp.float32),
            pltpu.SemaphoreType.DMA,
        ],
    )(_sc_gather_body)


def _gate_math(g, c):
    i_g = jax.nn.sigmoid(g[:, 0:H])
    f_g = jax.nn.sigmoid(g[:, H:2 * H])
    g_g = jnp.tanh(g[:, 2 * H:3 * H])
    o_g = jax.nn.sigmoid(g[:, 3 * H:4 * H])
    cn = f_g * c + i_g * g_g
    hn = o_g * jnp.tanh(cn)
    return hn, cn


def _lstm_body(x_ref, tok_ref, wih1_ref, whh1_ref, wih2_ref, whh2_ref,
               bih_ref, bhh_ref, gam_ref, bet_ref, ctx_ref, hf_ref, cf_ref,
               gates1_ref, gates2_ref, outs1_ref):
    mask = (tok_ref[...] != 0).astype(jnp.int32)
    lengths = jnp.sum(mask, axis=0)
    len_col = lengths.reshape(B, 1)
    gam = gam_ref[...].reshape(1, H)
    bet = bet_ref[...].reshape(1, H)

    wih1 = wih1_ref[...]
    whh1 = whh1_ref[...]
    wih2 = wih2_ref[...]
    whh2 = whh2_ref[...]
    bsum1 = (bih_ref[0] + bhh_ref[0]).reshape(1, G4)
    bsum2 = (bih_ref[1] + bhh_ref[1]).reshape(1, G4)

    def l1_step(t, ti, h1, c1):
        g = gates1_ref[pl.ds(pl.multiple_of(ti * B, B), B), :] + jnp.dot(
            h1.astype(jnp.bfloat16), whh1, preferred_element_type=jnp.float32)
        hn, cn = _gate_math(g, c1)
        valid = len_col > t
        outs1_ref[pl.ds(pl.multiple_of(ti * B, B), B), :] = (
            jnp.where(valid, hn, 0.0))
        return jnp.where(valid, hn, h1), jnp.where(valid, cn, c1)

    def l2_step(t2, ti, row, h2, c2):
        g = gates2_ref[pl.ds(pl.multiple_of(ti * B, B), B), :]
        g = g + jnp.dot(h2.astype(jnp.bfloat16), whh2,
                        preferred_element_type=jnp.float32)
        hn, cn = _gate_math(g, c2)
        valid = len_col > t2
        y2 = jnp.where(valid, hn, 0.0)
        s1 = jnp.sum(y2, axis=-1, keepdims=True)
        s2 = jnp.sum(y2 * y2, axis=-1, keepdims=True)
        mu = s1 * (1.0 / H)
        var = s2 * (1.0 / H) - mu * mu
        ln = (y2 - mu) * lax.rsqrt(var + 1e-5) * gam + bet
        ctx_ref[pl.ds(row, B), :] = jnp.where(valid, ln, 0.0)
        return jnp.where(valid, hn, h2), jnp.where(valid, cn, c2)

    z = jnp.zeros((B, H), jnp.float32)
    carry = (z, z, z, z)
    for ph in range(NCHUNK + 1):
        if ph < NCHUNK:
            xin = x_ref[pl.ds(ph * CHUNK * B, CHUNK * B), :]
            gates1_ref[...] = (
                jnp.dot(xin, wih1, preferred_element_type=jnp.float32)
                + bsum1)
        if ph > 0:
            gates2_ref[...] = (
                jnp.dot(outs1_ref[...], wih2,
                        preferred_element_type=jnp.float32) + bsum2)

        def step(ti, carry, _ph=ph):
            h1, c1, h2, c2 = carry
            if _ph < NCHUNK:
                h1, c1 = l1_step(_ph * CHUNK + ti, ti, h1, c1)
            if _ph > 0:
                t2 = (_ph - 1) * CHUNK + ti
                row = pl.multiple_of((_ph - 1) * CHUNK * B + ti * B, B)
                h2, c2 = l2_step(t2, ti, row, h2, c2)
            return h1, c1, h2, c2

        carry = lax.fori_loop(0, CHUNK, step, carry, unroll=4)
    h1, c1, h2, c2 = carry

    li = lengths[:, None]
    lj = lengths[None, :]
    ii = lax.broadcasted_iota(jnp.int32, (B, B), 0)
    jj = lax.broadcasted_iota(jnp.int32, (B, B), 1)
    before = jnp.logical_or(lj > li, jnp.logical_and(lj == li, jj < ii))
    rank = jnp.sum(before.astype(jnp.int32), axis=1)
    kk = lax.broadcasted_iota(jnp.int32, (B, B), 0)
    P = (rank[None, :] == kk).astype(jnp.float32)
    for l, (h, c) in enumerate(((h1, c1), (h2, c2))):
        hf_ref[l] = jnp.dot(P, h, preferred_element_type=jnp.float32)
        cf_ref[l] = jnp.dot(P, c, preferred_element_type=jnp.float32)


def _tc_lstm(x_flat, tokens, Wih1T, Whh1T, Wih2T, Whh2T, bih, bhh,
             gamma, beta):
    nlayer = bih.shape[0]
    return pl.pallas_call(
        _lstm_body,
        out_shape=[
            jax.ShapeDtypeStruct((T * B, H), jnp.float32),
            jax.ShapeDtypeStruct((nlayer, B, H), jnp.float32),
            jax.ShapeDtypeStruct((nlayer, B, H), jnp.float32),
        ],
        scratch_shapes=[
            pltpu.VMEM((CHUNK * B, G4), jnp.float32),
            pltpu.VMEM((CHUNK * B, G4), jnp.float32),
            pltpu.VMEM((CHUNK * B, H), jnp.float32),
        ],
        compiler_params=pltpu.CompilerParams(
            vmem_limit_bytes=120 * 1024 * 1024),
    )(x_flat, tokens, Wih1T, Whh1T, Wih2T, Whh2T, bih, bhh, gamma, beta)


def kernel(input, table, Wih, Whh, bih, bhh, gamma, beta):
    idx_flat = input.reshape(-1)
    x_flat = _sc_gather()(table, idx_flat)
    WihT = jnp.swapaxes(Wih, 1, 2)
    WhhT = jnp.swapaxes(Whh, 1, 2)
    ctx, hf, cf = _tc_lstm(x_flat, input, WihT[0],
                           WhhT[0].astype(jnp.bfloat16), WihT[1],
                           WhhT[1].astype(jnp.bfloat16),
                           bih, bhh, gamma, beta)
    return ctx.reshape(T, B, H), hf, cf


# speedup vs baseline: 14.4151x; 1.0340x over previous
import functools

import jax
import jax.numpy as jnp
from jax import lax
from jax.experimental import pallas as pl
from jax.experimental.pallas import tpu as pltpu
from jax.experimental.pallas import tpu_sc as plsc

T, B, H, V = 512, 8, 512, 32000
G4 = 4 * H
CHUNK = 64
NCHUNK = T // CHUNK
NC, NS = 2, 16
NW = NC * NS
ROWS_PER_W = (T * B) // NW


def _sc_gather_body(table_hbm, idx_hbm, out_hbm, idx_v, rows_v, sem):
    wid = lax.axis_index("s") * NC + lax.axis_index("c")
    base = wid * ROWS_PER_W
    pltpu.sync_copy(idx_hbm.at[pl.ds(base, ROWS_PER_W)], idx_v)
    pltpu.async_copy(table_hbm.at[idx_v], rows_v, sem).wait()
    pltpu.sync_copy(rows_v, out_hbm.at[pl.ds(base, ROWS_PER_W)])


@functools.cache
def _sc_gather():
    return functools.partial(
        pl.kernel,
        out_type=jax.ShapeDtypeStruct((T * B, H), jnp.float32),
        mesh=plsc.VectorSubcoreMesh(core_axis_name="c", subcore_axis_name="s"),
        scratch_types=[
            pltpu.VMEM((ROWS_PER_W,), jnp.int32),
            pltpu.VMEM((ROWS_PER_W, H), jnp.float32),
            pltpu.SemaphoreType.DMA,
        ],
    )(_sc_gather_body)


def _gate_math(g, c):
    i_g = jax.nn.sigmoid(g[:, 0:H])
    f_g = jax.nn.sigmoid(g[:, H:2 * H])
    g_g = jnp.tanh(g[:, 2 * H:3 * H])
    o_g = jax.nn.sigmoid(g[:, 3 * H:4 * H])
    cn = f_g * c + i_g * g_g
    hn = o_g * jnp.tanh(cn)
    return hn, cn


def _lstm_body(x_ref, tok_ref, wih1_ref, whh1_ref, wih2_ref, whh2_ref,
               bih_ref, bhh_ref, gam_ref, bet_ref, ctx_ref, hf_ref, cf_ref,
               gates1_ref, gates2_ref, outs1_ref):
    mask = (tok_ref[...] != 0).astype(jnp.int32)
    lengths = jnp.sum(mask, axis=0)
    len_col = lengths.reshape(B, 1)
    gam = gam_ref[...].reshape(1, H)
    bet = bet_ref[...].reshape(1, H)

    wih1 = wih1_ref[...]
    whh1 = whh1_ref[...]
    wih2 = wih2_ref[...]
    whh2 = whh2_ref[...]
    bsum1 = (bih_ref[0] + bhh_ref[0]).reshape(1, G4)
    bsum2 = (bih_ref[1] + bhh_ref[1]).reshape(1, G4)

    def l1_step(t, ti, h1, c1):
        g = gates1_ref[pl.ds(pl.multiple_of(ti * B, B), B), :] + jnp.dot(
            h1.astype(jnp.bfloat16), whh1, preferred_element_type=jnp.float32)
        hn, cn = _gate_math(g, c1)
        valid = len_col > t
        outs1_ref[pl.ds(pl.multiple_of(ti * B, B), B), :] = (
            jnp.where(valid, hn, 0.0))
        return jnp.where(valid, hn, h1), jnp.where(valid, cn, c1)

    def l2_step(t2, ti, row, h2, c2):
        g = gates2_ref[pl.ds(pl.multiple_of(ti * B, B), B), :]
        g = g + jnp.dot(h2.astype(jnp.bfloat16), whh2,
                        preferred_element_type=jnp.float32)
        hn, cn = _gate_math(g, c2)
        valid = len_col > t2
        y2 = jnp.where(valid, hn, 0.0)
        s1 = jnp.sum(y2, axis=-1, keepdims=True)
        s2 = jnp.sum(y2 * y2, axis=-1, keepdims=True)
        mu = s1 * (1.0 / H)
        var = s2 * (1.0 / H) - mu * mu
        ln = (y2 - mu) * lax.rsqrt(var + 1e-5) * gam + bet
        ctx_ref[pl.ds(row, B), :] = jnp.where(valid, ln, 0.0)
        return jnp.where(valid, hn, h2), jnp.where(valid, cn, c2)

    z = jnp.zeros((B, H), jnp.float32)
    carry = (z, z, z, z)
    for ph in range(NCHUNK + 1):
        if ph < NCHUNK:
            xin = x_ref[pl.ds(ph * CHUNK * B, CHUNK * B), :]
            gates1_ref[...] = (
                jnp.dot(xin, wih1, preferred_element_type=jnp.float32)
                + bsum1)
        if ph > 0:
            gates2_ref[...] = (
                jnp.dot(outs1_ref[...], wih2,
                        preferred_element_type=jnp.float32) + bsum2)

        def step(ti, carry, _ph=ph):
            h1, c1, h2, c2 = carry
            if _ph < NCHUNK:
                h1, c1 = l1_step(_ph * CHUNK + ti, ti, h1, c1)
            if _ph > 0:
                t2 = (_ph - 1) * CHUNK + ti
                row = pl.multiple_of((_ph - 1) * CHUNK * B + ti * B, B)
                h2, c2 = l2_step(t2, ti, row, h2, c2)
            return h1, c1, h2, c2

        carry = lax.fori_loop(0, CHUNK, step, carry, unroll=8)
    h1, c1, h2, c2 = carry

    li = lengths[:, None]
    lj = lengths[None, :]
    ii = lax.broadcasted_iota(jnp.int32, (B, B), 0)
    jj = lax.broadcasted_iota(jnp.int32, (B, B), 1)
    before = jnp.logical_or(lj > li, jnp.logical_and(lj == li, jj < ii))
    rank = jnp.sum(before.astype(jnp.int32), axis=1)
    kk = lax.broadcasted_iota(jnp.int32, (B, B), 0)
    P = (rank[None, :] == kk).astype(jnp.float32)
    for l, (h, c) in enumerate(((h1, c1), (h2, c2))):
        hf_ref[l] = jnp.dot(P, h, preferred_element_type=jnp.float32)
        cf_ref[l] = jnp.dot(P, c, preferred_element_type=jnp.float32)


def _tc_lstm(x_flat, tokens, Wih1T, Whh1T, Wih2T, Whh2T, bih, bhh,
             gamma, beta):
    nlayer = bih.shape[0]
    return pl.pallas_call(
        _lstm_body,
        out_shape=[
            jax.ShapeDtypeStruct((T * B, H), jnp.float32),
            jax.ShapeDtypeStruct((nlayer, B, H), jnp.float32),
            jax.ShapeDtypeStruct((nlayer, B, H), jnp.float32),
        ],
        scratch_shapes=[
            pltpu.VMEM((CHUNK * B, G4), jnp.float32),
            pltpu.VMEM((CHUNK * B, G4), jnp.float32),
            pltpu.VMEM((CHUNK * B, H), jnp.float32),
        ],
        compiler_params=pltpu.CompilerParams(
            vmem_limit_bytes=120 * 1024 * 1024),
    )(x_flat, tokens, Wih1T, Whh1T, Wih2T, Whh2T, bih, bhh, gamma, beta)


def kernel(input, table, Wih, Whh, bih, bhh, gamma, beta):
    idx_flat = input.reshape(-1)
    x_flat = _sc_gather()(table, idx_flat)
    WihT = jnp.swapaxes(Wih, 1, 2)
    WhhT = jnp.swapaxes(Whh, 1, 2)
    ctx, hf, cf = _tc_lstm(x_flat, input, WihT[0],
                           WhhT[0].astype(jnp.bfloat16), WihT[1],
                           WhhT[1].astype(jnp.bfloat16),
                           bih, bhh, gamma, beta)
    return ctx.reshape(T, B, H), hf, cf


# speedup vs baseline: 14.6752x; 1.0180x over previous
import functools

import jax
import jax.numpy as jnp
from jax import lax
from jax.experimental import pallas as pl
from jax.experimental.pallas import tpu as pltpu
from jax.experimental.pallas import tpu_sc as plsc

T, B, H, V = 512, 8, 512, 32000
G4 = 4 * H
CHUNK = 64
NCHUNK = T // CHUNK
NC, NS = 2, 16
NW = NC * NS
ROWS_PER_W = (T * B) // NW


def _sc_gather_body(table_hbm, idx_hbm, out_hbm, idx_v, rows_v, sem):
    wid = lax.axis_index("s") * NC + lax.axis_index("c")
    base = wid * ROWS_PER_W
    pltpu.sync_copy(idx_hbm.at[pl.ds(base, ROWS_PER_W)], idx_v)
    pltpu.async_copy(table_hbm.at[idx_v], rows_v, sem).wait()
    pltpu.sync_copy(rows_v, out_hbm.at[pl.ds(base, ROWS_PER_W)])


@functools.cache
def _sc_gather():
    return functools.partial(
        pl.kernel,
        out_type=jax.ShapeDtypeStruct((T * B, H), jnp.float32),
        mesh=plsc.VectorSubcoreMesh(core_axis_name="c", subcore_axis_name="s"),
        scratch_types=[
            pltpu.VMEM((ROWS_PER_W,), jnp.int32),
            pltpu.VMEM((ROWS_PER_W, H), jnp.float32),
            pltpu.SemaphoreType.DMA,
        ],
    )(_sc_gather_body)


def _gate_math(g, c):
    i_g = jax.nn.sigmoid(g[:, 0:H])
    f_g = jax.nn.sigmoid(g[:, H:2 * H])
    g_g = jnp.tanh(g[:, 2 * H:3 * H])
    o_g = jax.nn.sigmoid(g[:, 3 * H:4 * H])
    cn = f_g * c + i_g * g_g
    hn = o_g * jnp.tanh(cn)
    return hn, cn


def _lstm_body(x_ref, tok_ref, wih1_ref, whh1_ref, wih2_ref, whh2_ref,
               bih_ref, bhh_ref, gam_ref, bet_ref, ctx_ref, hf_ref, cf_ref,
               gates1_ref, gates2_ref, outs1_ref):
    mask = (tok_ref[...] != 0).astype(jnp.int32)
    lengths = jnp.sum(mask, axis=0)
    len_col = lengths.reshape(B, 1)
    gam = gam_ref[...].reshape(1, H)
    bet = bet_ref[...].reshape(1, H)

    wih1 = wih1_ref[...]
    whh1 = whh1_ref[...]
    wih2 = wih2_ref[...]
    whh2 = whh2_ref[...]
    bsum1 = (bih_ref[0] + bhh_ref[0]).reshape(1, G4)
    bsum2 = (bih_ref[1] + bhh_ref[1]).reshape(1, G4)

    def l1_step(t, ti, h1, c1):
        g = gates1_ref[pl.ds(pl.multiple_of(ti * B, B), B), :] + jnp.dot(
            h1.astype(jnp.bfloat16), whh1, preferred_element_type=jnp.float32)
        hn, cn = _gate_math(g, c1)
        valid = len_col > t
        outs1_ref[pl.ds(pl.multiple_of(ti * B, B), B), :] = (
            jnp.where(valid, hn, 0.0))
        return jnp.where(valid, hn, h1), jnp.where(valid, cn, c1)

    def l2_step(t2, ti, row, h2, c2):
        g = gates2_ref[pl.ds(pl.multiple_of(ti * B, B), B), :]
        g = g + jnp.dot(h2.astype(jnp.bfloat16), whh2,
                        preferred_element_type=jnp.float32)
        hn, cn = _gate_math(g, c2)
        valid = len_col > t2
        y2 = jnp.where(valid, hn, 0.0)
        s1 = jnp.sum(y2, axis=-1, keepdims=True)
        s2 = jnp.sum(y2 * y2, axis=-1, keepdims=True)
        mu = s1 * (1.0 / H)
        var = s2 * (1.0 / H) - mu * mu
        ln = (y2 - mu) * lax.rsqrt(var + 1e-5) * gam + bet
        ctx_ref[pl.ds(row, B), :] = jnp.where(valid, ln, 0.0)
        return jnp.where(valid, hn, h2), jnp.where(valid, cn, c2)

    z = jnp.zeros((B, H), jnp.float32)
    carry = (z, z, z, z)
    for ph in range(NCHUNK + 1):
        if ph < NCHUNK:
            xin = x_ref[pl.ds(ph * CHUNK * B, CHUNK * B), :]
            gates1_ref[...] = (
                jnp.dot(xin, wih1, preferred_element_type=jnp.float32)
                + bsum1)
        if ph > 0:
            gates2_ref[...] = (
                jnp.dot(outs1_ref[...], wih2,
                        preferred_element_type=jnp.float32) + bsum2)

        def step(ti, carry, _ph=ph):
            h1, c1, h2, c2 = carry
            if _ph < NCHUNK:
                h1, c1 = l1_step(_ph * CHUNK + ti, ti, h1, c1)
            if _ph > 0:
                t2 = (_ph - 1) * CHUNK + ti
                row = pl.multiple_of((_ph - 1) * CHUNK * B + ti * B, B)
                h2, c2 = l2_step(t2, ti, row, h2, c2)
            return h1, c1, h2, c2

        carry = lax.fori_loop(0, CHUNK, step, carry, unroll=16)
    h1, c1, h2, c2 = carry

    li = lengths[:, None]
    lj = lengths[None, :]
    ii = lax.broadcasted_iota(jnp.int32, (B, B), 0)
    jj = lax.broadcasted_iota(jnp.int32, (B, B), 1)
    before = jnp.logical_or(lj > li, jnp.logical_and(lj == li, jj < ii))
    rank = jnp.sum(before.astype(jnp.int32), axis=1)
    kk = lax.broadcasted_iota(jnp.int32, (B, B), 0)
    P = (rank[None, :] == kk).astype(jnp.float32)
    for l, (h, c) in enumerate(((h1, c1), (h2, c2))):
        hf_ref[l] = jnp.dot(P, h, preferred_element_type=jnp.float32)
        cf_ref[l] = jnp.dot(P, c, preferred_element_type=jnp.float32)


def _tc_lstm(x_flat, tokens, Wih1T, Whh1T, Wih2T, Whh2T, bih, bhh,
             gamma, beta):
    nlayer = bih.shape[0]
    return pl.pallas_call(
        _lstm_body,
        out_shape=[
            jax.ShapeDtypeStruct((T * B, H), jnp.float32),
            jax.ShapeDtypeStruct((nlayer, B, H), jnp.float32),
            jax.ShapeDtypeStruct((nlayer, B, H), jnp.float32),
        ],
        scratch_shapes=[
            pltpu.VMEM((CHUNK * B, G4), jnp.float32),
            pltpu.VMEM((CHUNK * B, G4), jnp.float32),
            pltpu.VMEM((CHUNK * B, H), jnp.float32),
        ],
        compiler_params=pltpu.CompilerParams(
            vmem_limit_bytes=120 * 1024 * 1024),
    )(x_flat, tokens, Wih1T, Whh1T, Wih2T, Whh2T, bih, bhh, gamma, beta)


def kernel(input, table, Wih, Whh, bih, bhh, gamma, beta):
    idx_flat = input.reshape(-1)
    x_flat = _sc_gather()(table, idx_flat)
    WihT = jnp.swapaxes(Wih, 1, 2)
    WhhT = jnp.swapaxes(Whh, 1, 2)
    ctx, hf, cf = _tc_lstm(x_flat, input, WihT[0],
                           WhhT[0].astype(jnp.bfloat16), WihT[1],
                           WhhT[1].astype(jnp.bfloat16),
                           bih, bhh, gamma, beta)
    return ctx.reshape(T, B, H), hf, cf


# speedup vs baseline: 14.7539x; 1.0054x over previous
import functools

import jax
import jax.numpy as jnp
from jax import lax
from jax.experimental import pallas as pl
from jax.experimental.pallas import tpu as pltpu
from jax.experimental.pallas import tpu_sc as plsc

T, B, H, V = 512, 8, 512, 32000
G4 = 4 * H
CHUNK = 32
NCHUNK = T // CHUNK
NC, NS = 2, 16
NW = NC * NS
ROWS_PER_W = (T * B) // NW


def _sc_gather_body(table_hbm, idx_hbm, out_hbm, idx_v, rows_v, sem):
    wid = lax.axis_index("s") * NC + lax.axis_index("c")
    base = wid * ROWS_PER_W
    pltpu.sync_copy(idx_hbm.at[pl.ds(base, ROWS_PER_W)], idx_v)
    pltpu.async_copy(table_hbm.at[idx_v], rows_v, sem).wait()
    pltpu.sync_copy(rows_v, out_hbm.at[pl.ds(base, ROWS_PER_W)])


@functools.cache
def _sc_gather():
    return functools.partial(
        pl.kernel,
        out_type=jax.ShapeDtypeStruct((T * B, H), jnp.float32),
        mesh=plsc.VectorSubcoreMesh(core_axis_name="c", subcore_axis_name="s"),
        scratch_types=[
            pltpu.VMEM((ROWS_PER_W,), jnp.int32),
            pltpu.VMEM((ROWS_PER_W, H), jnp.float32),
            pltpu.SemaphoreType.DMA,
        ],
    )(_sc_gather_body)


def _gate_math(g, c):
    i_g = jax.nn.sigmoid(g[:, 0:H])
    f_g = jax.nn.sigmoid(g[:, H:2 * H])
    g_g = jnp.tanh(g[:, 2 * H:3 * H])
    o_g = jax.nn.sigmoid(g[:, 3 * H:4 * H])
    cn = f_g * c + i_g * g_g
    hn = o_g * jnp.tanh(cn)
    return hn, cn


def _lstm_body(x_ref, tok_ref, wih1_ref, whh1_ref, wih2_ref, whh2_ref,
               bih_ref, bhh_ref, gam_ref, bet_ref, ctx_ref, hf_ref, cf_ref,
               gates1_ref, gates2_ref, outs1_ref):
    mask = (tok_ref[...] != 0).astype(jnp.int32)
    lengths = jnp.sum(mask, axis=0)
    len_col = lengths.reshape(B, 1)
    gam = gam_ref[...].reshape(1, H)
    bet = bet_ref[...].reshape(1, H)

    wih1 = wih1_ref[...]
    whh1 = whh1_ref[...]
    wih2 = wih2_ref[...]
    whh2 = whh2_ref[...]
    bsum1 = (bih_ref[0] + bhh_ref[0]).reshape(1, G4)
    bsum2 = (bih_ref[1] + bhh_ref[1]).reshape(1, G4)

    def l1_step(t, ti, h1, c1):
        g = gates1_ref[pl.ds(pl.multiple_of(ti * B, B), B), :] + jnp.dot(
            h1.astype(jnp.bfloat16), whh1, preferred_element_type=jnp.float32)
        hn, cn = _gate_math(g, c1)
        valid = len_col > t
        outs1_ref[pl.ds(pl.multiple_of(ti * B, B), B), :] = (
            jnp.where(valid, hn, 0.0))
        return jnp.where(valid, hn, h1), jnp.where(valid, cn, c1)

    def l2_step(t2, ti, row, h2, c2):
        g = gates2_ref[pl.ds(pl.multiple_of(ti * B, B), B), :]
        g = g + jnp.dot(h2.astype(jnp.bfloat16), whh2,
                        preferred_element_type=jnp.float32)
        hn, cn = _gate_math(g, c2)
        valid = len_col > t2
        y2 = jnp.where(valid, hn, 0.0)
        s1 = jnp.sum(y2, axis=-1, keepdims=True)
        s2 = jnp.sum(y2 * y2, axis=-1, keepdims=True)
        mu = s1 * (1.0 / H)
        var = s2 * (1.0 / H) - mu * mu
        ln = (y2 - mu) * lax.rsqrt(var + 1e-5) * gam + bet
        ctx_ref[pl.ds(row, B), :] = jnp.where(valid, ln, 0.0)
        return jnp.where(valid, hn, h2), jnp.where(valid, cn, c2)

    z = jnp.zeros((B, H), jnp.float32)
    carry = (z, z, z, z)
    for ph in range(NCHUNK + 1):
        if ph < NCHUNK:
            xin = x_ref[pl.ds(ph * CHUNK * B, CHUNK * B), :]
            gates1_ref[...] = (
                jnp.dot(xin, wih1, preferred_element_type=jnp.float32)
                + bsum1)
        if ph > 0:
            gates2_ref[...] = (
                jnp.dot(outs1_ref[...], wih2,
                        preferred_element_type=jnp.float32) + bsum2)

        def step(ti, carry, _ph=ph):
            h1, c1, h2, c2 = carry
            if _ph < NCHUNK:
                h1, c1 = l1_step(_ph * CHUNK + ti, ti, h1, c1)
            if _ph > 0:
                t2 = (_ph - 1) * CHUNK + ti
                row = pl.multiple_of((_ph - 1) * CHUNK * B + ti * B, B)
                h2, c2 = l2_step(t2, ti, row, h2, c2)
            return h1, c1, h2, c2

        carry = lax.fori_loop(0, CHUNK, step, carry, unroll=16)
    h1, c1, h2, c2 = carry

    li = lengths[:, None]
    lj = lengths[None, :]
    ii = lax.broadcasted_iota(jnp.int32, (B, B), 0)
    jj = lax.broadcasted_iota(jnp.int32, (B, B), 1)
    before = jnp.logical_or(lj > li, jnp.logical_and(lj == li, jj < ii))
    rank = jnp.sum(before.astype(jnp.int32), axis=1)
    kk = lax.broadcasted_iota(jnp.int32, (B, B), 0)
    P = (rank[None, :] == kk).astype(jnp.float32)
    for l, (h, c) in enumerate(((h1, c1), (h2, c2))):
        hf_ref[l] = jnp.dot(P, h, preferred_element_type=jnp.float32)
        cf_ref[l] = jnp.dot(P, c, preferred_element_type=jnp.float32)


def _tc_lstm(x_flat, tokens, Wih1T, Whh1T, Wih2T, Whh2T, bih, bhh,
             gamma, beta):
    nlayer = bih.shape[0]
    return pl.pallas_call(
        _lstm_body,
        out_shape=[
            jax.ShapeDtypeStruct((T * B, H), jnp.float32),
            jax.ShapeDtypeStruct((nlayer, B, H), jnp.float32),
            jax.ShapeDtypeStruct((nlayer, B, H), jnp.float32),
        ],
        scratch_shapes=[
            pltpu.VMEM((CHUNK * B, G4), jnp.float32),
            pltpu.VMEM((CHUNK * B, G4), jnp.float32),
            pltpu.VMEM((CHUNK * B, H), jnp.float32),
        ],
        compiler_params=pltpu.CompilerParams(
            vmem_limit_bytes=120 * 1024 * 1024),
    )(x_flat, tokens, Wih1T, Whh1T, Wih2T, Whh2T, bih, bhh, gamma, beta)


def kernel(input, table, Wih, Whh, bih, bhh, gamma, beta):
    idx_flat = input.reshape(-1)
    x_flat = _sc_gather()(table, idx_flat)
    WihT = jnp.swapaxes(Wih, 1, 2)
    WhhT = jnp.swapaxes(Whh, 1, 2)
    ctx, hf, cf = _tc_lstm(x_flat, input, WihT[0],
                           WhhT[0].astype(jnp.bfloat16), WihT[1],
                           WhhT[1].astype(jnp.bfloat16),
                           bih, bhh, gamma, beta)
    return ctx.reshape(T, B, H), hf, cf
